# baseline (speedup 1.0000x reference)
import jax
import jax.numpy as jnp
from jax.experimental import pallas as pl
from jax.experimental.pallas import tpu as pltpu

_PROB = 0.5

_LANES = 1024
_BLOCK_ROWS = 1024


def _body(x_ref, n_ref, o_ref):
    x = x_ref[...]
    o_ref[...] = jnp.where(n_ref[...] < _PROB, jnp.round(x), x)


def kernel(x, noise):
    orig_shape = x.shape
    total = x.size
    rows = total // _LANES
    xf = x.reshape(rows, _LANES)
    nf = noise.reshape(rows, _LANES)
    grid = (rows // _BLOCK_ROWS,)
    spec = pl.BlockSpec((_BLOCK_ROWS, _LANES), lambda i: (i, 0))
    out = pl.pallas_call(
        _body,
        grid=grid,
        in_specs=[spec, spec],
        out_specs=spec,
        out_shape=jax.ShapeDtypeStruct((rows, _LANES), jnp.float32),
        compiler_params=pltpu.CompilerParams(
            dimension_semantics=("parallel",),
        ),
    )(xf, nf)
    return out.reshape(orig_shape)


# speedup vs baseline: 1.4708x; 1.4708x over previous
import jax
import jax.numpy as jnp
from jax.experimental import pallas as pl
from jax.experimental.pallas import tpu as pltpu

_PROB = 0.5


def _body(x_ref, n_ref, o_ref):
    x = x_ref[...]
    o_ref[...] = jnp.where(n_ref[...] < _PROB, jnp.round(x), x)


def kernel(x, noise):
    B, C, H, W = x.shape
    bc = 64
    grid = (B, C // bc)
    spec = pl.BlockSpec((1, bc, H, W), lambda i, j: (i, j, 0, 0))
    return pl.pallas_call(
        _body,
        grid=grid,
        in_specs=[spec, spec],
        out_specs=spec,
        out_shape=jax.ShapeDtypeStruct(x.shape, x.dtype),
        compiler_params=pltpu.CompilerParams(
            dimension_semantics=("parallel", "parallel"),
        ),
    )(x, noise)


# speedup vs baseline: 10.7252x; 7.2920x over previous
import jax
import jax.numpy as jnp
from jax.experimental import pallas as pl
from jax.experimental.pallas import tpu as pltpu

_PROB = 0.5


def _body(x_ref, n_ref, o_ref):
    x = x_ref[...]
    o_ref[...] = jnp.where(n_ref[...] < _PROB, jnp.round(x), x)


def kernel(x, noise):
    B, C, H, W = x.shape
    xt = jnp.transpose(x, (0, 2, 3, 1))
    nt = jnp.transpose(noise, (0, 2, 3, 1))
    spec = pl.BlockSpec((1, H, W, C), lambda i: (i, 0, 0, 0))
    out = pl.pallas_call(
        _body,
        grid=(B,),
        in_specs=[spec, spec],
        out_specs=spec,
        out_shape=jax.ShapeDtypeStruct((B, H, W, C), x.dtype),
        compiler_params=pltpu.CompilerParams(
            dimension_semantics=("parallel",),
        ),
    )(xt, nt)
    return jnp.transpose(out, (0, 3, 1, 2))


# speedup vs baseline: 10.7376x; 1.0012x over previous
import jax
import jax.numpy as jnp
from jax.experimental import pallas as pl
from jax.experimental.pallas import tpu as pltpu

_PROB = 0.5


def _body(x_ref, n_ref, o_ref):
    x = x_ref[...]
    o_ref[...] = jnp.where(n_ref[...] < _PROB, jnp.round(x), x)


def kernel(x, noise):
    B, C, H, W = x.shape
    xt = jnp.transpose(x, (0, 2, 3, 1))
    nt = jnp.transpose(noise, (0, 2, 3, 1))
    spec = pl.BlockSpec((2, H, W, C), lambda i: (i, 0, 0, 0))
    out = pl.pallas_call(
        _body,
        grid=(B // 2,),
        in_specs=[spec, spec],
        out_specs=spec,
        out_shape=jax.ShapeDtypeStruct((B, H, W, C), x.dtype),
        compiler_params=pltpu.CompilerParams(
            dimension_semantics=("parallel",),
        ),
    )(xt, nt)
    return jnp.transpose(out, (0, 3, 1, 2))
